# baseline (speedup 1.0000x reference)
import functools

import jax
import jax.numpy as jnp
from jax import lax
from jax.experimental import pallas as pl
from jax.experimental.pallas import tpu as pltpu
from jax.experimental.pallas import tpu_sc as plsc

EMBEDDING_DIM = 32
SEQ = 20
CROWS = 64
NBUF = 2


@functools.lru_cache(maxsize=None)
def _make_sc_gather(R: int, D: int):
    info = plsc.get_sparse_core_info()
    NC, NS = info.num_cores, info.num_subcores
    NW = NC * NS
    rows_per_w = R // NW
    n_chunks = rows_per_w // CROWS
    chunk = CROWS * SEQ
    mesh = plsc.VectorSubcoreMesh(core_axis_name="c", subcore_axis_name="s")

    @functools.partial(
        pl.kernel,
        mesh=mesh,
        out_type=jax.ShapeDtypeStruct((R, SEQ, D), jnp.float32),
        scratch_types=[
            pltpu.VMEM((n_chunks, chunk), jnp.int32),
            pltpu.VMEM((NBUF, chunk, D), jnp.float32),
            pltpu.SemaphoreType.DMA,
            pltpu.SemaphoreType.DMA,
            pltpu.SemaphoreType.DMA,
            pltpu.SemaphoreType.DMA,
        ],
        compiler_params=pltpu.CompilerParams(use_tc_tiling_on_sc=False),
    )
    def sc_gather(idx_hbm, table_hbm, out_hbm, idx_v, rows_v, gs0, gs1, ss0, ss1):
        wid = lax.axis_index("s") * NC + lax.axis_index("c")
        row0 = wid * rows_per_w
        gsem = (gs0, gs1)
        ssem = (ss0, ss1)

        pltpu.sync_copy(idx_hbm.at[wid], idx_v)

        gathers = [None] * n_chunks
        stores = [[] for _ in range(n_chunks)]

        def start_stores(i):
            b = i % NBUF
            base = row0 + i * CROWS
            for s in range(SEQ):
                stores[i].append(
                    pltpu.async_copy(
                        rows_v.at[b, pl.ds(s * CROWS, CROWS)],
                        out_hbm.at[pl.ds(base, CROWS), s],
                        ssem[b],
                    )
                )

        for i in range(n_chunks):
            b = i % NBUF
            if i >= NBUF:
                for cp in stores[i - NBUF]:
                    cp.wait()
            gathers[i] = pltpu.async_copy(
                table_hbm.at[idx_v.at[i]], rows_v.at[b], gsem[b]
            )
            if i >= 1:
                gathers[i - 1].wait()
                start_stores(i - 1)
        last = n_chunks - 1
        gathers[last].wait()
        start_stores(last)
        for i in (last - 1, last):
            for cp in stores[i]:
                cp.wait()

    return sc_gather


def kernel(cards, table):
    R, S = cards.shape
    info = plsc.get_sparse_core_info()
    NW = info.num_cores * info.num_subcores
    n_chunks = (R // NW) // CROWS
    idx = (
        cards.astype(jnp.int32)
        .reshape(NW, n_chunks, CROWS, S)
        .transpose(0, 1, 3, 2)
        .reshape(NW, n_chunks, CROWS * S)
    )
    return _make_sc_gather(R, EMBEDDING_DIM)(idx, table)


# speedup vs baseline: 4.3412x; 4.3412x over previous
import functools

import jax
import jax.numpy as jnp
from jax import lax
from jax.experimental import pallas as pl
from jax.experimental.pallas import tpu as pltpu
from jax.experimental.pallas import tpu_sc as plsc

D = 32
SEQ = 20
VPAD = 64
NBUF = 2


@functools.lru_cache(maxsize=None)
def _make_sc_embed(R: int):
    info = plsc.get_sparse_core_info()
    NC, NS, L = info.num_cores, info.num_subcores, info.num_lanes
    NW = NC * NS
    rows_w = R // NW
    n_bhi = rows_w // 128
    n_grp = rows_w // L
    mesh = plsc.VectorSubcoreMesh(core_axis_name="c", subcore_axis_name="s")

    @functools.partial(
        pl.kernel,
        mesh=mesh,
        out_type=jax.ShapeDtypeStruct((SEQ, D // 8, R // 128, 8, 128), jnp.float32),
        scratch_types=[
            pltpu.VMEM((D * VPAD,), jnp.float32),
            pltpu.VMEM((SEQ, rows_w), jnp.int32),
            pltpu.VMEM((NBUF, D // 8, n_bhi, 8, 128), jnp.float32),
            pltpu.SemaphoreType.DMA,
            pltpu.SemaphoreType.DMA,
        ],
        compiler_params=pltpu.CompilerParams(
            use_tc_tiling_on_sc=False, needs_layout_passes=False
        ),
    )
    def sc_embed(idx_hbm, tab_hbm, out_hbm, tab_v, idx_v, buf, ss0, ss1):
        wid = lax.axis_index("s") * NC + lax.axis_index("c")
        ssem = (ss0, ss1)

        pltpu.sync_copy(tab_hbm, tab_v)
        pltpu.sync_copy(idx_hbm.at[wid], idx_v)

        def make_compute(s, bs):
            def body(g, carry):
                g_hi = g // 8
                g_lo16 = (g % 8) * L
                idxv = idx_v[s, pl.ds(g * L, L)]
                for c in range(D):
                    val = plsc.load_gather(tab_v, [idxv + c * VPAD])
                    buf[bs, c // 8, g_hi, c % 8, pl.ds(g_lo16, L)] = val
                return carry

            return body

        copies = [None] * SEQ
        for s in range(SEQ):
            bs = s % NBUF
            if s >= NBUF:
                for cp in copies[s - NBUF]:
                    cp.wait()
            lax.fori_loop(0, n_grp, make_compute(s, bs), 0)
            copies[s] = [
                pltpu.async_copy(
                    buf.at[bs, c_hi],
                    out_hbm.at[s, c_hi, pl.ds(wid * n_bhi, n_bhi)],
                    ssem[bs],
                )
                for c_hi in range(D // 8)
            ]
        for s in (SEQ - NBUF, SEQ - 1):
            for cp in copies[s]:
                cp.wait()

    return sc_embed


def kernel(cards, table):
    R, S = cards.shape
    info = plsc.get_sparse_core_info()
    NW = info.num_cores * info.num_subcores
    idx = cards.astype(jnp.int32).reshape(NW, R // NW, S).transpose(0, 2, 1)
    tab_t = jnp.zeros((D, VPAD), jnp.float32).at[:, : table.shape[0]].set(table.T)
    phys = _make_sc_embed(R)(idx, tab_t.reshape(D * VPAD))
    return phys.transpose(2, 4, 0, 1, 3).reshape(R, S, D)


# speedup vs baseline: 7.1437x; 1.6456x over previous
import functools

import jax
import jax.numpy as jnp
from jax import lax
from jax.experimental import pallas as pl
from jax.experimental.pallas import tpu as pltpu
from jax.experimental.pallas import tpu_sc as plsc

D = 32
SEQ = 20
VPAD = 64
NBUF = 2


@functools.lru_cache(maxsize=None)
def _make_sc_embed(R: int):
    info = plsc.get_sparse_core_info()
    NC, NS, L = info.num_cores, info.num_subcores, info.num_lanes
    NW = NC * NS
    rows_w = R // NW
    n_bhi = rows_w // 128
    n_grp = rows_w // L
    mesh = plsc.VectorSubcoreMesh(core_axis_name="c", subcore_axis_name="s")

    @functools.partial(
        pl.kernel,
        mesh=mesh,
        out_type=jax.ShapeDtypeStruct((SEQ, D // 8, R // 128, 8, 128), jnp.float32),
        scratch_types=[
            pltpu.VMEM((D * VPAD,), jnp.float32),
            pltpu.VMEM((SEQ, rows_w), jnp.int32),
            pltpu.VMEM((NBUF, D // 8, n_bhi, 8, 128), jnp.float32),
            pltpu.SemaphoreType.DMA,
            pltpu.SemaphoreType.DMA,
        ],
        compiler_params=pltpu.CompilerParams(
            use_tc_tiling_on_sc=False, needs_layout_passes=False
        ),
    )
    def sc_embed(idx_hbm, tab_hbm, out_hbm, tab_v, idx_v, buf, ss0, ss1):
        wid = lax.axis_index("s") * NC + lax.axis_index("c")
        ssem = (ss0, ss1)

        pltpu.sync_copy(tab_hbm, tab_v)
        pltpu.sync_copy(idx_hbm.at[wid], idx_v)

        def make_compute(s, bs):
            def body(g, carry):
                g_hi = g // 8
                g_lo16 = (g % 8) * L
                idxv = idx_v[s, pl.ds(g * L, L)]
                vals = [plsc.load_gather(tab_v, [idxv + c * VPAD]) for c in range(D)]
                for c in range(D):
                    buf[bs, c // 8, g_hi, c % 8, pl.ds(g_lo16, L)] = vals[c]
                return carry

            return body

        copies = [None] * SEQ
        for s in range(SEQ):
            bs = s % NBUF
            if s >= NBUF:
                for cp in copies[s - NBUF]:
                    cp.wait()
            lax.fori_loop(0, n_grp, make_compute(s, bs), 0)
            copies[s] = [
                pltpu.async_copy(
                    buf.at[bs, c_hi],
                    out_hbm.at[s, c_hi, pl.ds(wid * n_bhi, n_bhi)],
                    ssem[bs],
                )
                for c_hi in range(D // 8)
            ]
        for s in (SEQ - NBUF, SEQ - 1):
            for cp in copies[s]:
                cp.wait()

    return sc_embed


def kernel(cards, table):
    R, S = cards.shape
    info = plsc.get_sparse_core_info()
    NW = info.num_cores * info.num_subcores
    idx = cards.astype(jnp.int32).reshape(NW, R // NW, S).transpose(0, 2, 1)
    tab_t = jnp.zeros((D, VPAD), jnp.float32).at[:, : table.shape[0]].set(table.T)
    phys = _make_sc_embed(R)(idx, tab_t.reshape(D * VPAD))
    return phys.transpose(2, 4, 0, 1, 3).reshape(R, S, D)


# speedup vs baseline: 7.7086x; 1.0791x over previous
import functools

import jax
import jax.numpy as jnp
from jax import lax
from jax.experimental import pallas as pl
from jax.experimental.pallas import tpu as pltpu
from jax.experimental.pallas import tpu_sc as plsc

D = 32
SEQ = 20
VPAD = 64
NBUF = 2


@functools.lru_cache(maxsize=None)
def _make_sc_embed(R: int):
    info = plsc.get_sparse_core_info()
    NC, NS, L = info.num_cores, info.num_subcores, info.num_lanes
    NW = NC * NS
    rows_w = R // NW
    n_bhi = rows_w // 128
    n_grp = rows_w // L
    mesh = plsc.VectorSubcoreMesh(core_axis_name="c", subcore_axis_name="s")

    @functools.partial(
        pl.kernel,
        mesh=mesh,
        out_type=jax.ShapeDtypeStruct((SEQ, D // 8, R // 128, 8, 128), jnp.float32),
        scratch_types=[
            pltpu.VMEM((D * VPAD,), jnp.float32),
            pltpu.VMEM((SEQ, rows_w), jnp.int32),
            pltpu.VMEM((NBUF, D // 8, n_bhi, 8, 128), jnp.float32),
            pltpu.SemaphoreType.DMA,
            pltpu.SemaphoreType.DMA,
        ],
        compiler_params=pltpu.CompilerParams(
            use_tc_tiling_on_sc=False, needs_layout_passes=False
        ),
    )
    def sc_embed(idx_hbm, tab_hbm, out_hbm, tab_v, idx_v, buf, ss0, ss1):
        wid = lax.axis_index("s") * NC + lax.axis_index("c")
        ssem = (ss0, ss1)

        pltpu.sync_copy(tab_hbm, tab_v)
        pltpu.sync_copy(idx_hbm.at[wid], idx_v)

        def make_compute(s, bs):
            def body(g, carry):
                g_hi = g // 8
                g_lo16 = (g % 8) * L
                idxv = idx_v[s, pl.ds(g * L, L)]
                for c0 in range(0, D, 16):
                    vals = [
                        plsc.load_gather(tab_v, [idxv + c * VPAD])
                        for c in range(c0, c0 + 16)
                    ]
                    for j, c in enumerate(range(c0, c0 + 16)):
                        buf[bs, c // 8, g_hi, c % 8, pl.ds(g_lo16, L)] = vals[j]
                return carry

            return body

        copies = [None] * SEQ
        for s in range(SEQ):
            bs = s % NBUF
            if s >= NBUF:
                for cp in copies[s - NBUF]:
                    cp.wait()
            lax.fori_loop(0, n_grp, make_compute(s, bs), 0)
            copies[s] = [
                pltpu.async_copy(
                    buf.at[bs, c_hi],
                    out_hbm.at[s, c_hi, pl.ds(wid * n_bhi, n_bhi)],
                    ssem[bs],
                )
                for c_hi in range(D // 8)
            ]
        for s in (SEQ - NBUF, SEQ - 1):
            for cp in copies[s]:
                cp.wait()

    return sc_embed


def kernel(cards, table):
    R, S = cards.shape
    info = plsc.get_sparse_core_info()
    NW = info.num_cores * info.num_subcores
    idx = cards.astype(jnp.int32).reshape(NW, R // NW, S).transpose(0, 2, 1)
    tab_t = jnp.zeros((D, VPAD), jnp.float32).at[:, : table.shape[0]].set(table.T)
    phys = _make_sc_embed(R)(idx, tab_t.reshape(D * VPAD))
    return phys.transpose(2, 4, 0, 1, 3).reshape(R, S, D)


# speedup vs baseline: 8.8173x; 1.1438x over previous
import functools

import jax
import jax.numpy as jnp
from jax import lax
from jax.experimental import pallas as pl
from jax.experimental.pallas import tpu as pltpu
from jax.experimental.pallas import tpu_sc as plsc

D = 32
SEQ = 20
VPAD = 64
NBUF = 2


@functools.lru_cache(maxsize=None)
def _make_sc_embed(R: int):
    info = plsc.get_sparse_core_info()
    NC, NS, L = info.num_cores, info.num_subcores, info.num_lanes
    NW = NC * NS
    rows_w = R // NW
    n_bhi = rows_w // 128
    n_grp = rows_w // L
    mesh = plsc.VectorSubcoreMesh(core_axis_name="c", subcore_axis_name="s")

    @functools.partial(
        pl.kernel,
        mesh=mesh,
        out_type=jax.ShapeDtypeStruct((SEQ, D // 8, R // 128, 8, 128), jnp.float32),
        scratch_types=[
            pltpu.VMEM((D * VPAD * L,), jnp.float32),
            pltpu.VMEM((SEQ, rows_w), jnp.int32),
            pltpu.VMEM((NBUF, D // 8, n_bhi, 8, 128), jnp.float32),
            pltpu.SemaphoreType.DMA,
            pltpu.SemaphoreType.DMA,
        ],
        compiler_params=pltpu.CompilerParams(
            use_tc_tiling_on_sc=False, needs_layout_passes=False
        ),
    )
    def sc_embed(idx_hbm, tab_hbm, out_hbm, tab_v, idx_v, buf, ss0, ss1):
        wid = lax.axis_index("s") * NC + lax.axis_index("c")
        ssem = (ss0, ss1)

        pltpu.sync_copy(tab_hbm, tab_v)
        pltpu.sync_copy(idx_hbm.at[wid], idx_v)

        def make_compute(s, bs):
            def body(g, carry):
                g_hi = g // 8
                g_lo16 = (g % 8) * L
                idx16 = idx_v[s, pl.ds(g * L, L)] * L + lax.iota(jnp.int32, L)
                for c0 in range(0, D, 16):
                    vals = [
                        plsc.load_gather(tab_v, [idx16 + c * (VPAD * L)])
                        for c in range(c0, c0 + 16)
                    ]
                    for j, c in enumerate(range(c0, c0 + 16)):
                        buf[bs, c // 8, g_hi, c % 8, pl.ds(g_lo16, L)] = vals[j]
                return carry

            return body

        copies = [None] * SEQ
        for s in range(SEQ):
            bs = s % NBUF
            if s >= NBUF:
                for cp in copies[s - NBUF]:
                    cp.wait()
            lax.fori_loop(0, n_grp, make_compute(s, bs), 0)
            copies[s] = [
                pltpu.async_copy(
                    buf.at[bs, c_hi],
                    out_hbm.at[s, c_hi, pl.ds(wid * n_bhi, n_bhi)],
                    ssem[bs],
                )
                for c_hi in range(D // 8)
            ]
        for s in (SEQ - NBUF, SEQ - 1):
            for cp in copies[s]:
                cp.wait()

    return sc_embed


def kernel(cards, table):
    R, S = cards.shape
    info = plsc.get_sparse_core_info()
    NW = info.num_cores * info.num_subcores
    idx = cards.astype(jnp.int32).reshape(NW, R // NW, S).transpose(0, 2, 1)
    tab_t = jnp.zeros((D, VPAD), jnp.float32).at[:, : table.shape[0]].set(table.T)
    tab_r = jnp.broadcast_to(tab_t.reshape(D * VPAD, 1), (D * VPAD, 16))
    phys = _make_sc_embed(R)(idx, tab_r.reshape(D * VPAD * 16))
    return phys.transpose(2, 4, 0, 1, 3).reshape(R, S, D)


# speedup vs baseline: 9.1740x; 1.0405x over previous
import functools

import jax
import jax.numpy as jnp
from jax import lax
from jax.experimental import pallas as pl
from jax.experimental.pallas import tpu as pltpu
from jax.experimental.pallas import tpu_sc as plsc

D = 32
SEQ = 20
VPAD = 64
NBUF = 2


@functools.lru_cache(maxsize=None)
def _make_sc_embed(R: int):
    info = plsc.get_sparse_core_info()
    NC, NS, L = info.num_cores, info.num_subcores, info.num_lanes
    NW = NC * NS
    rows_w = R // NW
    n_bhi = rows_w // 128
    n_grp = rows_w // L
    mesh = plsc.VectorSubcoreMesh(core_axis_name="c", subcore_axis_name="s")

    @functools.partial(
        pl.kernel,
        mesh=mesh,
        out_type=jax.ShapeDtypeStruct((SEQ, D // 8, R // 128, 8, 128), jnp.float32),
        scratch_types=[
            pltpu.VMEM((D * VPAD * L,), jnp.float32),
            pltpu.VMEM((SEQ, rows_w), jnp.int32),
            pltpu.VMEM((NBUF, D // 8, n_bhi, 8, 128), jnp.float32),
            pltpu.SemaphoreType.DMA((NBUF,)),
        ],
        compiler_params=pltpu.CompilerParams(
            use_tc_tiling_on_sc=False, needs_layout_passes=False
        ),
    )
    def sc_embed(idx_hbm, tab_hbm, out_hbm, tab_v, idx_v, buf, ssem):
        wid = lax.axis_index("s") * NC + lax.axis_index("c")

        pltpu.sync_copy(tab_hbm, tab_v)
        pltpu.sync_copy(idx_hbm.at[wid], idx_v)

        def drain(bs):
            for c_hi in range(D // 8):
                pltpu.make_async_copy(
                    out_hbm.at[0, c_hi, pl.ds(wid * n_bhi, n_bhi)],
                    buf.at[bs, c_hi],
                    ssem.at[bs],
                ).wait()

        def s_body(s, carry):
            bs = s % NBUF

            @pl.when(s >= NBUF)
            def _():
                drain(bs)

            def body(g, carry_g):
                g_hi = g // 8
                g_lo16 = (g % 8) * L
                idx16 = idx_v[s, pl.ds(g * L, L)] * L + lax.iota(jnp.int32, L)
                for c0 in range(0, D, 16):
                    vals = [
                        plsc.load_gather(tab_v, [idx16 + c * (VPAD * L)])
                        for c in range(c0, c0 + 16)
                    ]
                    for j, c in enumerate(range(c0, c0 + 16)):
                        buf[bs, c // 8, g_hi, c % 8, pl.ds(g_lo16, L)] = vals[j]
                return carry_g

            lax.fori_loop(0, n_grp, body, 0)
            for c_hi in range(D // 8):
                pltpu.async_copy(
                    buf.at[bs, c_hi],
                    out_hbm.at[s, c_hi, pl.ds(wid * n_bhi, n_bhi)],
                    ssem.at[bs],
                )
            return carry

        lax.fori_loop(0, SEQ, s_body, 0)
        drain(SEQ % NBUF)
        drain((SEQ + 1) % NBUF)

    return sc_embed


def kernel(cards, table):
    R, S = cards.shape
    info = plsc.get_sparse_core_info()
    NW = info.num_cores * info.num_subcores
    idx = cards.astype(jnp.int32).reshape(NW, R // NW, S).transpose(0, 2, 1)
    tab_t = jnp.zeros((D, VPAD), jnp.float32).at[:, : table.shape[0]].set(table.T)
    tab_r = jnp.broadcast_to(tab_t.reshape(D * VPAD, 1), (D * VPAD, 16))
    phys = _make_sc_embed(R)(idx, tab_r.reshape(D * VPAD * 16))
    return phys.transpose(2, 4, 0, 1, 3).reshape(R, S, D)


# speedup vs baseline: 9.3019x; 1.0139x over previous
import functools

import jax
import jax.numpy as jnp
from jax import lax
from jax.experimental import pallas as pl
from jax.experimental.pallas import tpu as pltpu
from jax.experimental.pallas import tpu_sc as plsc

D = 32
SEQ = 20
VPAD = 64
NBUF = 2
W = 8


@functools.lru_cache(maxsize=None)
def _make_sc_embed(R: int):
    info = plsc.get_sparse_core_info()
    NC, NS, L = info.num_cores, info.num_subcores, info.num_lanes
    NW = NC * NS
    rows_w = R // NW
    n_bhi = rows_w // 128
    n_grp = rows_w // L
    mesh = plsc.VectorSubcoreMesh(core_axis_name="c", subcore_axis_name="s")

    @functools.partial(
        pl.kernel,
        mesh=mesh,
        out_type=jax.ShapeDtypeStruct((SEQ, D // 8, R // 128, 8, 128), jnp.float32),
        scratch_types=[
            pltpu.VMEM((D * VPAD * L,), jnp.float32),
            pltpu.VMEM((SEQ, rows_w), jnp.int32),
            pltpu.VMEM((NBUF, D // 8, n_bhi, 8, 128), jnp.float32),
            pltpu.SemaphoreType.DMA((NBUF,)),
        ],
        compiler_params=pltpu.CompilerParams(
            use_tc_tiling_on_sc=False, needs_layout_passes=False
        ),
    )
    def sc_embed(idx_hbm, tab_hbm, out_hbm, tab_v, idx_v, buf, ssem):
        wid = lax.axis_index("s") * NC + lax.axis_index("c")

        pltpu.sync_copy(tab_hbm, tab_v)
        pltpu.sync_copy(idx_hbm.at[wid], idx_v)

        def drain(bs):
            for c_hi in range(D // 8):
                pltpu.make_async_copy(
                    out_hbm.at[0, c_hi, pl.ds(wid * n_bhi, n_bhi)],
                    buf.at[bs, c_hi],
                    ssem.at[bs],
                ).wait()

        def s_body(s, carry):
            bs = s % NBUF

            @pl.when(s >= NBUF)
            def _():
                drain(bs)

            def body(g, carry_g):
                g_hi = g // 8
                g_lo16 = (g % 8) * L
                idx16 = idx_v[s, pl.ds(g * L, L)] * L + lax.iota(jnp.int32, L)

                def gather_wave(c0):
                    return [
                        plsc.load_gather(
                            tab_v.at[pl.ds(c * (VPAD * L), VPAD * L)], [idx16]
                        )
                        for c in range(c0, c0 + W)
                    ]

                def store_wave(c0, vals):
                    for j, c in enumerate(range(c0, c0 + W)):
                        buf[bs, c // 8, g_hi, c % 8, pl.ds(g_lo16, L)] = vals[j]

                W_ = D // W
                pend_c0, pend = 0, gather_wave(0)
                for k in range(1, W_):
                    nxt = gather_wave(k * W)
                    store_wave(pend_c0, pend)
                    pend_c0, pend = k * W, nxt
                store_wave(pend_c0, pend)
                return carry_g

            lax.fori_loop(0, n_grp, body, 0)
            for c_hi in range(D // 8):
                pltpu.async_copy(
                    buf.at[bs, c_hi],
                    out_hbm.at[s, c_hi, pl.ds(wid * n_bhi, n_bhi)],
                    ssem.at[bs],
                )
            return carry

        lax.fori_loop(0, SEQ, s_body, 0)
        drain(SEQ % NBUF)
        drain((SEQ + 1) % NBUF)

    return sc_embed


def kernel(cards, table):
    R, S = cards.shape
    info = plsc.get_sparse_core_info()
    NW = info.num_cores * info.num_subcores
    idx = cards.astype(jnp.int32).reshape(NW, R // NW, S).transpose(0, 2, 1)
    tab_t = jnp.zeros((D, VPAD), jnp.float32).at[:, : table.shape[0]].set(table.T)
    tab_r = jnp.broadcast_to(tab_t.reshape(D * VPAD, 1), (D * VPAD, 16))
    phys = _make_sc_embed(R)(idx, tab_r.reshape(D * VPAD * 16))
    return phys.transpose(2, 4, 0, 1, 3).reshape(R, S, D)


# speedup vs baseline: 10.4774x; 1.1264x over previous
import functools

import jax
import jax.numpy as jnp
from jax import lax
from jax.experimental import pallas as pl
from jax.experimental.pallas import tpu as pltpu
from jax.experimental.pallas import tpu_sc as plsc

D = 32
SEQ = 20
VPAD = 64
NBUF = 2
W = 8


@functools.lru_cache(maxsize=None)
def _make_sc_embed(R: int):
    info = plsc.get_sparse_core_info()
    NC, NS, L = info.num_cores, info.num_subcores, info.num_lanes
    NW = NC * NS
    rows_w = R // NW
    n_bhi = rows_w // 128
    n_grp = rows_w // L
    mesh = plsc.VectorSubcoreMesh(core_axis_name="c", subcore_axis_name="s")

    @functools.partial(
        pl.kernel,
        mesh=mesh,
        out_type=jax.ShapeDtypeStruct((SEQ, D // 8, R // 128, 8, 128), jnp.float32),
        scratch_types=[
            pltpu.VMEM((D * VPAD * L,), jnp.float32),
            pltpu.VMEM((SEQ, rows_w), jnp.int32),
            pltpu.VMEM((NBUF, D // 8, n_bhi, 8, 128), jnp.float32),
            pltpu.SemaphoreType.DMA((NBUF,)),
        ],
        compiler_params=pltpu.CompilerParams(
            use_tc_tiling_on_sc=False, needs_layout_passes=False
        ),
    )
    def sc_embed(idx_hbm, tab_hbm, out_hbm, tab_v, idx_v, buf, ssem):
        wid = lax.axis_index("s") * NC + lax.axis_index("c")

        pltpu.sync_copy(tab_hbm, tab_v)
        pltpu.sync_copy(idx_hbm.at[wid], idx_v)

        def drain(bs):
            for c_hi in range(D // 8):
                pltpu.make_async_copy(
                    out_hbm.at[0, c_hi, pl.ds(wid * n_bhi, n_bhi)],
                    buf.at[bs, c_hi],
                    ssem.at[bs],
                ).wait()

        def s_body(s, carry):
            bs = s % NBUF

            @pl.when(s >= NBUF)
            def _():
                drain(bs)

            def body(g, carry_g):
                g_hi = g // 8
                g_lo16 = (g % 8) * L
                idx16 = idx_v[s, pl.ds(g * L, L)] * L + lax.iota(jnp.int32, L)

                def gather(c):
                    return plsc.load_gather(
                        tab_v.at[pl.ds(c * (VPAD * L), VPAD * L)], [idx16]
                    )

                def store(c, val):
                    buf[bs, c // 8, g_hi, c % 8, pl.ds(g_lo16, L)] = val

                prev = None
                for k in range(D // W):
                    cur = []
                    for j in range(W):
                        cur.append(gather(k * W + j))
                        if prev is not None:
                            store((k - 1) * W + j, prev[j])
                    prev = cur
                for j in range(W):
                    store(D - W + j, prev[j])
                return carry_g

            lax.fori_loop(0, n_grp, body, 0)
            for c_hi in range(D // 8):
                pltpu.async_copy(
                    buf.at[bs, c_hi],
                    out_hbm.at[s, c_hi, pl.ds(wid * n_bhi, n_bhi)],
                    ssem.at[bs],
                )
            return carry

        lax.fori_loop(0, SEQ, s_body, 0)
        drain(SEQ % NBUF)
        drain((SEQ + 1) % NBUF)

    return sc_embed


def kernel(cards, table):
    R, S = cards.shape
    info = plsc.get_sparse_core_info()
    NW = info.num_cores * info.num_subcores
    idx = cards.astype(jnp.int32).reshape(NW, R // NW, S).transpose(0, 2, 1)
    tab_t = jnp.zeros((D, VPAD), jnp.float32).at[:, : table.shape[0]].set(table.T)
    tab_r = jnp.broadcast_to(tab_t.reshape(D * VPAD, 1), (D * VPAD, 16))
    phys = _make_sc_embed(R)(idx, tab_r.reshape(D * VPAD * 16))
    return phys.transpose(2, 4, 0, 1, 3).reshape(R, S, D)


# speedup vs baseline: 11.0953x; 1.0590x over previous
import functools

import jax
import jax.numpy as jnp
from jax import lax
from jax.experimental import pallas as pl
from jax.experimental.pallas import tpu as pltpu
from jax.experimental.pallas import tpu_sc as plsc

D = 32
SEQ = 20
VPAD = 64
NBUF = 2
W = 8


@functools.lru_cache(maxsize=None)
def _make_sc_embed(R: int):
    info = plsc.get_sparse_core_info()
    NC, NS, L = info.num_cores, info.num_subcores, info.num_lanes
    NW = NC * NS
    rows_w = R // NW
    n_bhi = rows_w // 128
    n_grp = rows_w // L
    mesh = plsc.VectorSubcoreMesh(core_axis_name="c", subcore_axis_name="s")

    @functools.partial(
        pl.kernel,
        mesh=mesh,
        out_type=jax.ShapeDtypeStruct((SEQ, D // 8, R // 128, 8, 128), jnp.float32),
        scratch_types=[
            pltpu.VMEM((D * VPAD * L,), jnp.float32),
            pltpu.VMEM((SEQ, rows_w), jnp.int32),
            pltpu.VMEM((NBUF, D // 8, n_bhi, 8, 128), jnp.float32),
            pltpu.SemaphoreType.DMA((NBUF,)),
        ],
        compiler_params=pltpu.CompilerParams(
            use_tc_tiling_on_sc=False, needs_layout_passes=False
        ),
    )
    def sc_embed(idx_hbm, tab_hbm, out_hbm, tab_v, idx_v, buf, ssem):
        wid = lax.axis_index("s") * NC + lax.axis_index("c")

        pltpu.sync_copy(tab_hbm, tab_v)
        pltpu.sync_copy(idx_hbm.at[:, pl.ds(wid * rows_w, rows_w)], idx_v)

        def drain(bs):
            for c_hi in range(D // 8):
                pltpu.make_async_copy(
                    out_hbm.at[0, c_hi, pl.ds(wid * n_bhi, n_bhi)],
                    buf.at[bs, c_hi],
                    ssem.at[bs],
                ).wait()

        def s_body(s, carry):
            bs = s % NBUF

            @pl.when(s >= NBUF)
            def _():
                drain(bs)

            def load_idx16(g):
                return idx_v[s, pl.ds(g * L, L)] * L + lax.iota(jnp.int32, L)

            def gather(c, idx16):
                return plsc.load_gather(
                    tab_v.at[pl.ds(c * (VPAD * L), VPAD * L)], [idx16]
                )

            def store(c, g_hi, g_lo16, val):
                buf[bs, c // 8, g_hi, c % 8, pl.ds(g_lo16, L)] = val

            NWAVE = D // W

            def body(g, prev):
                pg = g - 1
                pg_hi, pg_lo16 = pg // 8, (pg % 8) * L
                g_hi, g_lo16 = g // 8, (g % 8) * L
                idx16 = load_idx16(g)
                for k in range(NWAVE):
                    cur = []
                    for j in range(W):
                        cur.append(gather(k * W + j, idx16))
                        if k == 0:
                            store(D - W + j, pg_hi, pg_lo16, prev[j])
                        else:
                            store((k - 1) * W + j, g_hi, g_lo16, prev[j])
                    prev = tuple(cur)
                return prev

            idx16 = load_idx16(0)
            prev = None
            for k in range(NWAVE):
                cur = []
                for j in range(W):
                    cur.append(gather(k * W + j, idx16))
                    if prev is not None:
                        store((k - 1) * W + j, 0, 0, prev[j])
                prev = tuple(cur)
            prev = lax.fori_loop(1, n_grp, body, prev)
            lg = n_grp - 1
            for j in range(W):
                store(D - W + j, lg // 8, (lg % 8) * L, prev[j])
            for c_hi in range(D // 8):
                pltpu.async_copy(
                    buf.at[bs, c_hi],
                    out_hbm.at[s, c_hi, pl.ds(wid * n_bhi, n_bhi)],
                    ssem.at[bs],
                )
            return carry

        lax.fori_loop(0, SEQ, s_body, 0)
        drain(SEQ % NBUF)
        drain((SEQ + 1) % NBUF)

    return sc_embed


def kernel(cards, table):
    R, S = cards.shape
    info = plsc.get_sparse_core_info()
    NW = info.num_cores * info.num_subcores
    idx = cards.astype(jnp.int32).T
    tab_t = jnp.pad(table.T, ((0, 0), (0, VPAD - table.shape[0])))
    tab_r = jnp.broadcast_to(tab_t.reshape(D * VPAD, 1), (D * VPAD, 16))
    phys = _make_sc_embed(R)(idx, tab_r.reshape(D * VPAD * 16))
    return phys.transpose(2, 4, 0, 1, 3).reshape(R, S, D)
